# Initial kernel scaffold; baseline (speedup 1.0000x reference)
import jax
import jax.numpy as jnp
from jax.experimental import pallas as pl


def kernel(x, edge_index, batch, Wq1, bq1, Wk1, bk1, Wv1, bv1, Ws1, bs1, Wq2, bq2, Wk2, bk2, Wv2, bv2, Ws2, bs2, Wq3, bq3, Wk3, bk3, Wv3, bv3, Ws3, bs3, Wg, bg, Wf, bf):
    raise NotImplementedError("write your pallas kernel here")


# speedup vs baseline: 3.5026x; 3.5026x over previous
import functools

import jax
import jax.numpy as jnp
import numpy as np
from jax.experimental import pallas as pl

_N = 10000
_E = 160000
_H = 8
_NG = 16

_BM = 1000
_BE = 8000


def _mm_kernel(x_ref, w_ref, b_ref, o_ref):
    o_ref[...] = (
        jnp.dot(x_ref[...], w_ref[...], preferred_element_type=jnp.float32)
        + b_ref[...]
    )


def _matmul(x, w, b, bm):
    m, k = x.shape
    n = w.shape[1]
    return pl.pallas_call(
        _mm_kernel,
        grid=(m // bm,),
        in_specs=[
            pl.BlockSpec((bm, k), lambda i: (i, 0)),
            pl.BlockSpec((k, n), lambda i: (0, 0)),
            pl.BlockSpec((1, n), lambda i: (0, 0)),
        ],
        out_specs=pl.BlockSpec((bm, n), lambda i: (i, 0)),
        out_shape=jax.ShapeDtypeStruct((m, n), jnp.float32),
    )(x, w, b.reshape(1, n))


def _mm_add_elu_kernel(x_ref, w_ref, b_ref, a_ref, o_ref):
    z = (
        jnp.dot(x_ref[...], w_ref[...], preferred_element_type=jnp.float32)
        + b_ref[...]
        + a_ref[...]
    )
    o_ref[...] = jnp.where(z > 0, z, jnp.exp(jnp.minimum(z, 0.0)) - 1.0)


def _matmul_add_elu(x, w, b, acc, bm):
    m, k = x.shape
    n = w.shape[1]
    return pl.pallas_call(
        _mm_add_elu_kernel,
        grid=(m // bm,),
        in_specs=[
            pl.BlockSpec((bm, k), lambda i: (i, 0)),
            pl.BlockSpec((k, n), lambda i: (0, 0)),
            pl.BlockSpec((1, n), lambda i: (0, 0)),
            pl.BlockSpec((bm, n), lambda i: (i, 0)),
        ],
        out_specs=pl.BlockSpec((bm, n), lambda i: (i, 0)),
        out_shape=jax.ShapeDtypeStruct((m, n), jnp.float32),
    )(x, w, b.reshape(1, n), acc)


def _logit_kernel(qd_ref, ks_ref, o_ref, *, heads, c, scale):
    cols = []
    for h in range(heads):
        q = qd_ref[:, h * c:(h + 1) * c]
        k = ks_ref[:, h * c:(h + 1) * c]
        cols.append(jnp.sum(q * k, axis=1, keepdims=True))
    o_ref[...] = jnp.concatenate(cols, axis=1) * scale


_BE_WIDE = {4096: 320, 2048: 640, 1024: 1280}


def _edge_logits(qd, ks, c):
    e = qd.shape[0]
    be = _BE_WIDE[_H * c]
    scale = 1.0 / np.sqrt(c)
    return pl.pallas_call(
        functools.partial(_logit_kernel, heads=_H, c=c, scale=scale),
        grid=(e // be,),
        in_specs=[
            pl.BlockSpec((be, _H * c), lambda i: (i, 0)),
            pl.BlockSpec((be, _H * c), lambda i: (i, 0)),
        ],
        out_specs=pl.BlockSpec((be, _H), lambda i: (i, 0)),
        out_shape=jax.ShapeDtypeStruct((e, _H), jnp.float32),
    )(qd, ks)


def _exp_kernel(l_ref, m_ref, o_ref):
    o_ref[...] = jnp.exp(l_ref[...] - m_ref[...])


def _edge_exp(logit, m_e):
    e = logit.shape[0]
    return pl.pallas_call(
        _exp_kernel,
        grid=(e // _BE,),
        in_specs=[
            pl.BlockSpec((_BE, _H), lambda i: (i, 0)),
            pl.BlockSpec((_BE, _H), lambda i: (i, 0)),
        ],
        out_specs=pl.BlockSpec((_BE, _H), lambda i: (i, 0)),
        out_shape=jax.ShapeDtypeStruct((e, _H), jnp.float32),
    )(logit, m_e)


def _weighted_v_kernel(ex_ref, den_ref, vs_ref, o_ref, *, heads, c):
    alpha = ex_ref[...] / (den_ref[...] + 1e-16)
    acc = alpha[:, 0:1] * vs_ref[:, 0:c]
    for h in range(1, heads):
        acc = acc + alpha[:, h:h + 1] * vs_ref[:, h * c:(h + 1) * c]
    o_ref[...] = acc * (1.0 / heads)


def _edge_weighted_v(ex, den_e, vs, c):
    e = ex.shape[0]
    be = _BE_WIDE[_H * c]
    return pl.pallas_call(
        functools.partial(_weighted_v_kernel, heads=_H, c=c),
        grid=(e // be,),
        in_specs=[
            pl.BlockSpec((be, _H), lambda i: (i, 0)),
            pl.BlockSpec((be, _H), lambda i: (i, 0)),
            pl.BlockSpec((be, _H * c), lambda i: (i, 0)),
        ],
        out_specs=pl.BlockSpec((be, c), lambda i: (i, 0)),
        out_shape=jax.ShapeDtypeStruct((e, c), jnp.float32),
    )(ex, den_e, vs)


def _gate_kernel(h_ref, w_ref, b_ref, o_ref):
    o_ref[...] = jnp.sum(h_ref[...] * w_ref[...], axis=1, keepdims=True) + b_ref[...]


def _gate(hx, wg, bg):
    m, k = hx.shape
    return pl.pallas_call(
        _gate_kernel,
        grid=(m // _BM,),
        in_specs=[
            pl.BlockSpec((_BM, k), lambda i: (i, 0)),
            pl.BlockSpec((1, k), lambda i: (0, 0)),
            pl.BlockSpec((1, 1), lambda i: (0, 0)),
        ],
        out_specs=pl.BlockSpec((_BM, 1), lambda i: (i, 0)),
        out_shape=jax.ShapeDtypeStruct((m, 1), jnp.float32),
    )(hx, wg.reshape(1, k), bg.reshape(1, 1))


def _scale_rows_kernel(h_ref, a_ref, o_ref):
    o_ref[...] = h_ref[...] * a_ref[...]


def _scale_rows(hx, a):
    m, k = hx.shape
    return pl.pallas_call(
        _scale_rows_kernel,
        grid=(m // _BM,),
        in_specs=[
            pl.BlockSpec((_BM, k), lambda i: (i, 0)),
            pl.BlockSpec((_BM, 1), lambda i: (i, 0)),
        ],
        out_specs=pl.BlockSpec((_BM, k), lambda i: (i, 0)),
        out_shape=jax.ShapeDtypeStruct((m, k), jnp.float32),
    )(hx, a.reshape(m, 1))


def _final_mm_kernel(g_ref, w_ref, b_ref, o_ref):
    o_ref[...] = (
        jnp.dot(g_ref[...], w_ref[...], preferred_element_type=jnp.float32)
        + b_ref[...]
    )


def _final_mm(g, w, b):
    m, k = g.shape
    n = w.shape[1]
    return pl.pallas_call(
        _final_mm_kernel,
        in_specs=[
            pl.BlockSpec((m, k), lambda: (0, 0)),
            pl.BlockSpec((k, n), lambda: (0, 0)),
            pl.BlockSpec((1, n), lambda: (0, 0)),
        ],
        out_specs=pl.BlockSpec((m, n), lambda: (0, 0)),
        out_shape=jax.ShapeDtypeStruct((m, n), jnp.float32),
    )(g, w, b.reshape(1, n))


def _layer(x, src, dst, Wq, bq, Wk, bk, Wv, bv, Ws, bs, c):
    n = x.shape[0]
    q = _matmul(x, Wq, bq, _BM)
    k = _matmul(x, Wk, bk, _BM)
    v = _matmul(x, Wv, bv, _BM)

    qd = jnp.take(q, dst, axis=0)
    ks = jnp.take(k, src, axis=0)
    logit = _edge_logits(qd, ks, c)

    m = jax.ops.segment_max(logit, dst, num_segments=n)
    m = jnp.where(jnp.isfinite(m), m, 0.0)
    ex = _edge_exp(logit, jnp.take(m, dst, axis=0))
    den = jax.ops.segment_sum(ex, dst, num_segments=n)

    vs = jnp.take(v, src, axis=0)
    wv = _edge_weighted_v(ex, jnp.take(den, dst, axis=0), vs, c)
    acc = jax.ops.segment_sum(wv, dst, num_segments=n)

    return _matmul_add_elu(x, Ws, bs, acc, _BM)


def kernel(x, edge_index, batch, Wq1, bq1, Wk1, bk1, Wv1, bv1, Ws1, bs1, Wq2, bq2, Wk2, bk2, Wv2, bv2, Ws2, bs2, Wq3, bq3, Wk3, bk3, Wv3, bv3, Ws3, bs3, Wg, bg, Wf, bf):
    src = edge_index[0]
    dst = edge_index[1]
    h = _layer(x, src, dst, Wq1, bq1, Wk1, bk1, Wv1, bv1, Ws1, bs1, 512)
    h = _layer(h, src, dst, Wq2, bq2, Wk2, bk2, Wv2, bv2, Ws2, bs2, 256)
    h = _layer(h, src, dst, Wq3, bq3, Wk3, bk3, Wv3, bv3, Ws3, bs3, 128)

    gate = _gate(h, Wg[:, 0], bg)[:, 0]
    gm = jax.ops.segment_max(gate, batch, num_segments=_NG)
    gm = jnp.where(jnp.isfinite(gm), gm, 0.0)
    gex = jnp.exp(gate - jnp.take(gm, batch, axis=0))
    gden = jax.ops.segment_sum(gex, batch, num_segments=_NG)
    a = gex / (jnp.take(gden, batch, axis=0) + 1e-16)

    hw = _scale_rows(h, a)
    g = jax.ops.segment_sum(hw, batch, num_segments=_NG)
    return _final_mm(g, Wf, bf)
